# Initial kernel scaffold; baseline (speedup 1.0000x reference)
import jax
import jax.numpy as jnp
from jax.experimental import pallas as pl


def kernel(x, y, ANs_position, ANs_neighbours):
    raise NotImplementedError("write your pallas kernel here")


# speedup vs baseline: 1.5982x; 1.5982x over previous
import jax
import jax.numpy as jnp
from jax.experimental import pallas as pl
from jax.experimental.pallas import tpu as pltpu

B = 16384
C = 1000
A = 512
K = 10
R = 256
G = B // R


def _body(x_ref, y_ref, pos_ref, neigh_ref, out_ref):
    pid = pl.program_id(0)
    xb = x_ref[...]
    yb = y_ref[0, 0, :]
    posf = pos_ref[...].astype(jnp.float32)
    nbf = neigh_ref[...].astype(jnp.float32)

    col = jax.lax.broadcasted_iota(jnp.int32, (R, C), 1)
    ymask = (col == yb[:, None])

    m = jnp.max(xb, axis=1, keepdims=True)
    e = jnp.exp(xb - m)
    s = jnp.sum(e, axis=1)

    posv = jnp.sum(jnp.where(ymask, posf, 0.0), axis=1)
    is_anchor = posv >= 0.0
    spf = jnp.maximum(posv, 0.0)[:, None]

    arow = jax.lax.broadcasted_iota(jnp.int32, (R, A), 1)
    onehot = (arow == spf.astype(jnp.int32)).astype(jnp.float32)
    nks = jnp.dot(onehot, nbf, preferred_element_type=jnp.float32)
    nki = nks.astype(jnp.int32)

    wn = jnp.zeros((R, C), jnp.float32)
    for k in range(K):
        wn = wn + (col == nki[:, k][:, None]).astype(jnp.float32)
    anchorf = is_anchor.astype(jnp.float32)[:, None]
    wt = ymask.astype(jnp.float32) + anchorf * wn

    num = jnp.sum(e * wt, axis=1)
    loss = jnp.log(s) - jnp.log(num)
    total = jnp.sum(loss)

    @pl.when(pid == 0)
    def _():
        out_ref[0, 0] = 0.0

    out_ref[0, 0] += total


def kernel(x, y, ANs_position, ANs_neighbours):
    y3 = y.reshape(G, 1, R)
    pos2 = ANs_position.reshape(1, C)
    out = pl.pallas_call(
        _body,
        grid=(G,),
        in_specs=[
            pl.BlockSpec((R, C), lambda i: (i, 0)),
            pl.BlockSpec((1, 1, R), lambda i: (i, 0, 0)),
            pl.BlockSpec((1, C), lambda i: (0, 0)),
            pl.BlockSpec((A, K), lambda i: (0, 0)),
        ],
        out_specs=pl.BlockSpec(memory_space=pltpu.MemorySpace.SMEM),
        out_shape=jax.ShapeDtypeStruct((1, 1), jnp.float32),
    )(x, y3, pos2, ANs_neighbours)
    return out[0, 0] / B
